# baseline (speedup 1.0000x reference)
import functools
import jax, jax.numpy as jnp
from jax import lax
from jax.experimental import pallas as pl
from jax.experimental.pallas import tpu as pltpu
from jax.experimental.pallas import tpu_sc as plsc

_NC, _NS = 2, 16
_NW = _NC * _NS
_TOTAL = 16 * 3 * 512 * 512
_PER_W = _TOTAL // _NW
_CHUNK = 16384
_NBUF = 7
_NCH = _PER_W // _CHUNK

@functools.partial(
    pl.kernel,
    out_type=jax.ShapeDtypeStruct((_TOTAL // _CHUNK, _CHUNK), jnp.float32),
    mesh=plsc.VectorSubcoreMesh(core_axis_name="c", subcore_axis_name="s"),
    scratch_types=[
        pltpu.VMEM((_NBUF, _CHUNK), jnp.float32),
        pltpu.SemaphoreType.DMA((_NBUF,)),
        pltpu.SemaphoreType.DMA((_NBUF,)),
    ],
)
def _sc_copy(x_hbm, out_hbm, buf, isem, osem):
    wid = lax.axis_index("s") * _NC + lax.axis_index("c")
    base = wid * _NCH

    def cin(i, b):
        return pltpu.async_copy(
            x_hbm.at[pl.ds(base + i, 1)], buf.at[pl.ds(b, 1)], isem.at[b])

    def cout(i, b):
        return pltpu.async_copy(
            buf.at[pl.ds(b, 1)], out_hbm.at[pl.ds(base + i, 1)], osem.at[b])

    ins, outs = {}, {}
    for i in range(_NBUF):
        ins[i] = cin(i, i)
    for i in range(_NCH):
        b = i % _NBUF
        ins[i].wait()
        outs[i] = cout(i, b)
        j = i + _NBUF
        if j < _NCH:
            outs[i].wait()
            ins[j] = cin(j, b)
    for i in range(max(_NCH - _NBUF, 0), _NCH):
        outs[i].wait()

def kernel(x):
    return _sc_copy(x.reshape(_TOTAL // _CHUNK, _CHUNK)).reshape(x.shape)
